# baseline (speedup 1.0000x reference)
import functools

import jax
import jax.numpy as jnp
from jax import lax
from jax.experimental import pallas as pl
from jax.experimental.pallas import tpu as pltpu
from jax.experimental.pallas import tpu_sc as plsc

R = 16
N = 2048
B = 4096
X = 8192
NB = 8
L = 16
HALF = N // 2
NG = 8
GSZ = HALF // NG
CH = GSZ // L

_mesh = plsc.VectorSubcoreMesh(core_axis_name="c", subcore_axis_name="s")


@functools.partial(
    pl.kernel,
    out_type=jax.ShapeDtypeStruct((N,), jnp.int32),
    mesh=_mesh,
    compiler_params=pltpu.CompilerParams(
        needs_layout_passes=False, use_tc_tiling_on_sc=False),
    scratch_types=[
        pltpu.VMEM((X,), jnp.int32),
        pltpu.VMEM((B,), jnp.int32),
        pltpu.VMEM((HALF, NB), jnp.int32),
        pltpu.VMEM((NG, GSZ), jnp.int32),
        pltpu.VMEM((HALF,), jnp.int32),
        pltpu.VMEM((NG, GSZ, L), jnp.float32),
        pltpu.VMEM((16, 64), jnp.int32),
        pltpu.VMEM((16,), jnp.int32),
        pltpu.VMEM((16, 64), jnp.int32),
        pltpu.VMEM((64,), jnp.int32),
        pltpu.VMEM_SHARED((16, 64), jnp.int32),
        pltpu.SemaphoreType.DMA,
    ],
)
def _ensemble_ram_sc(x_hbm, proj_hbm, conn_hbm, mem_hbm, out_hbm,
                     x_v, proj_v, conn_v, idx_v, addr_v, rows_v, bits_v,
                     rowidx_v, red_v, outst_v, shared, sem):
    cid = lax.axis_index("c")
    sid = lax.axis_index("s")
    lane = lax.iota(jnp.int32, L)

    pltpu.sync_copy(x_hbm, x_v)
    pltpu.sync_copy(proj_hbm.at[sid], proj_v)
    pltpu.sync_copy(conn_hbm.at[sid, pl.ds(cid * HALF, HALF)], conn_v)

    nrow_base = (sid * N + cid * HALF) * 16

    def addr_body(g, _):
        for i in range(CH):
            nb = g * GSZ + i * L
            rowvec = nb + lane
            addr = jnp.zeros((L,), jnp.int32)
            for b in range(NB):
                c = plsc.load_gather(conn_v, [rowvec, jnp.full((L,), b, jnp.int32)])
                w = plsc.load_gather(proj_v, [c])
                bit = plsc.load_gather(x_v, [w])
                addr = addr + bit * (1 << b)
            addr_v[pl.ds(nb, L)] = addr
            blk = nrow_base + (nb + lane) * 16 + jnp.right_shift(addr, 4)
            idx_v[g, pl.ds(i * L, L)] = blk
        return _

    lax.fori_loop(0, NG, addr_body, None)

    copies = [
        pltpu.async_copy(mem_hbm.at[idx_v.at[g]], rows_v.at[g], sem)
        for g in range(NG)
    ]
    for c in copies:
        c.wait()

    def sel_body(g, _):
        gvec = jnp.full((L,), 0, jnp.int32) + g
        for i in range(CH):
            nb = g * GSZ + i * L
            addr = addr_v[pl.ds(nb, L)]
            rowvec = i * L + lane
            lanevec = jnp.bitwise_and(addr, 15)
            val = plsc.load_gather(rows_v, [gvec, rowvec, lanevec])
            bit = jnp.where(val > 0.5, 1, 0).astype(jnp.int32)
            bits_v[g * 2 + i // 4, pl.ds((i % 4) * L, L)] = bit
        return _

    lax.fori_loop(0, NG, sel_body, None)

    rowidx_v[...] = lane

    @pl.when(sid == 0)
    def _():
        pltpu.sync_copy(bits_v, shared)

    plsc.subcore_barrier()

    @pl.when(sid != 0)
    def _():
        pltpu.sync_copy(bits_v, shared.at[rowidx_v], add=True)

    plsc.subcore_barrier()
    pltpu.sync_copy(shared, red_v)
    W = HALF // R
    for gg in range(W // L):
        acc = red_v[sid, pl.ds(gg * L, L)]
        outst_v[pl.ds(gg * L, L)] = jnp.where(acc > R // 2, 1, 0).astype(jnp.int32)
    pltpu.sync_copy(outst_v, out_hbm.at[pl.ds(cid * HALF + sid * W, W)])


def kernel(x, projections, conn, memory):
    mem16 = memory.reshape(R * N * 16, 16)
    out = _ensemble_ram_sc(x, projections, conn, mem16)
    return out.astype(jnp.uint8)


# speedup vs baseline: 1.3220x; 1.3220x over previous
import functools

import jax
import jax.numpy as jnp
from jax import lax
from jax.experimental import pallas as pl
from jax.experimental.pallas import tpu as pltpu
from jax.experimental.pallas import tpu_sc as plsc

R = 16
N = 2048
B = 4096
X = 8192
NB = 8
L = 16
HALF = N // 2
NG = 8
GSZ = HALF // NG
CH = GSZ // L

_mesh = plsc.VectorSubcoreMesh(core_axis_name="c", subcore_axis_name="s")


@functools.partial(
    pl.kernel,
    out_type=jax.ShapeDtypeStruct((N,), jnp.int32),
    mesh=_mesh,
    compiler_params=pltpu.CompilerParams(
        needs_layout_passes=False, use_tc_tiling_on_sc=False),
    scratch_types=[
        pltpu.VMEM((X,), jnp.int32),
        pltpu.VMEM((B,), jnp.int32),
        pltpu.VMEM((HALF, NB), jnp.int32),
        pltpu.VMEM((NG, GSZ), jnp.int32),
        pltpu.VMEM((HALF,), jnp.int32),
        pltpu.VMEM((NG, GSZ, L), jnp.float32),
        pltpu.VMEM((16, 64), jnp.int32),
        pltpu.VMEM((16,), jnp.int32),
        pltpu.VMEM((16, 64), jnp.int32),
        pltpu.VMEM((64,), jnp.int32),
        pltpu.VMEM_SHARED((16, 64), jnp.int32),
        pltpu.SemaphoreType.DMA,
    ],
)
def _ensemble_ram_sc(x_hbm, proj_hbm, conn_hbm, mem_hbm, out_hbm,
                     x_v, proj_v, conn_v, idx_v, addr_v, rows_v, bits_v,
                     rowidx_v, red_v, outst_v, shared, sem):
    cid = lax.axis_index("c")
    sid = lax.axis_index("s")
    lane = lax.iota(jnp.int32, L)

    pltpu.sync_copy(x_hbm, x_v)
    pltpu.sync_copy(proj_hbm.at[sid], proj_v)
    pltpu.sync_copy(conn_hbm.at[sid, pl.ds(cid * HALF, HALF)], conn_v)

    q_base = sid * N + cid * HALF

    def addr_body(g, _):
        for i in range(CH):
            nb = g * GSZ + i * L
            rowvec = nb + lane
            addr = jnp.zeros((L,), jnp.int32)
            for b in range(NB):
                c = plsc.load_gather(conn_v, [rowvec, jnp.full((L,), b, jnp.int32)])
                w = plsc.load_gather(proj_v, [c])
                bit = plsc.load_gather(x_v, [w])
                addr = addr + bit * (1 << b)
            addr_v[pl.ds(nb, L)] = addr
            q = q_base + rowvec
            blk = (jnp.right_shift(q, 3) * 128
                   + jnp.right_shift(addr, 7) * 64
                   + jnp.bitwise_and(q, 7) * 8
                   + jnp.bitwise_and(jnp.right_shift(addr, 4), 7))
            idx_v[g, pl.ds(i * L, L)] = blk
        return _

    lax.fori_loop(0, NG, addr_body, None)

    copies = [
        pltpu.async_copy(mem_hbm.at[idx_v.at[g]], rows_v.at[g], sem)
        for g in range(NG)
    ]
    for c in copies:
        c.wait()

    def sel_body(g, _):
        gvec = jnp.full((L,), 0, jnp.int32) + g
        for i in range(CH):
            nb = g * GSZ + i * L
            addr = addr_v[pl.ds(nb, L)]
            rowvec = i * L + lane
            lanevec = jnp.bitwise_and(addr, 15)
            val = plsc.load_gather(rows_v, [gvec, rowvec, lanevec])
            bit = jnp.where(val > 0.5, 1, 0).astype(jnp.int32)
            bits_v[g * 2 + i // 4, pl.ds((i % 4) * L, L)] = bit
        return _

    lax.fori_loop(0, NG, sel_body, None)

    rowidx_v[...] = lane

    @pl.when(sid == 0)
    def _():
        pltpu.sync_copy(bits_v, shared)

    plsc.subcore_barrier()

    @pl.when(sid != 0)
    def _():
        pltpu.sync_copy(bits_v, shared.at[rowidx_v], add=True)

    plsc.subcore_barrier()
    pltpu.sync_copy(shared, red_v)
    W = HALF // R
    for gg in range(W // L):
        acc = red_v[sid, pl.ds(gg * L, L)]
        outst_v[pl.ds(gg * L, L)] = jnp.where(acc > R // 2, 1, 0).astype(jnp.int32)
    pltpu.sync_copy(outst_v, out_hbm.at[pl.ds(cid * HALF + sid * W, W)])


def kernel(x, projections, conn, memory):
    mem16 = (memory.reshape(R * N // 8, 8, 2, 128)
             .transpose(0, 2, 1, 3).reshape(R * N * 16, 16))
    out = _ensemble_ram_sc(x, projections, conn, mem16)
    return out.astype(jnp.uint8)


# speedup vs baseline: 1.3509x; 1.0219x over previous
import functools

import jax
import jax.numpy as jnp
from jax import lax
from jax.experimental import pallas as pl
from jax.experimental.pallas import tpu as pltpu
from jax.experimental.pallas import tpu_sc as plsc

R = 16
N = 2048
B = 4096
X = 8192
NB = 8
E = 2 ** NB
L = 16
HALF = N // 2
NG = 8
GSZ = HALF // NG
CH = GSZ // L

_mesh = plsc.VectorSubcoreMesh(core_axis_name="c", subcore_axis_name="s")


@functools.partial(
    pl.kernel,
    out_type=jax.ShapeDtypeStruct((N,), jnp.int32),
    mesh=_mesh,
    compiler_params=pltpu.CompilerParams(
        needs_layout_passes=False, use_tc_tiling_on_sc=True),
    scratch_types=[
        pltpu.VMEM((X,), jnp.int32),
        pltpu.VMEM((32, 128), jnp.int32),
        pltpu.VMEM((64, 128), jnp.int32),
        pltpu.VMEM((2, GSZ, E), jnp.float32),
        pltpu.VMEM((16, 128), jnp.int32),
        pltpu.VMEM((16,), jnp.int32),
        pltpu.VMEM((16, 128), jnp.int32),
        pltpu.VMEM((GSZ,), jnp.int32),
        pltpu.VMEM_SHARED((16, 128), jnp.int32),
        pltpu.SemaphoreType.DMA,
        pltpu.SemaphoreType.DMA,
    ],
)
def _ensemble_ram_sc(x_hbm, proj_hbm, conn_hbm, mem_hbm, out_hbm,
                     x_v, proj_v, conn_v, rows_v, bits_v,
                     rowidx_v, red_v, outst_v, shared, sem_in, sem_mem):
    cid = lax.axis_index("c")
    sid = lax.axis_index("s")
    lane = lax.iota(jnp.int32, L)
    n0 = cid * HALF

    mem_copies = {}
    for g in range(2):
        mem_copies[g] = pltpu.async_copy(
            mem_hbm.at[sid, pl.ds(n0 + g * GSZ, GSZ)], rows_v.at[g], sem_mem)
    in_copies = [
        pltpu.async_copy(x_hbm, x_v, sem_in),
        pltpu.async_copy(proj_hbm.at[sid], proj_v, sem_in),
        pltpu.async_copy(conn_hbm.at[sid, pl.ds(cid * 64, 64)], conn_v, sem_in),
    ]
    for c in in_copies:
        c.wait()

    zero = jnp.zeros((L,), jnp.int32)
    for t in range(8, 16):
        for k in range(8):
            bits_v[t, pl.ds(k * L, L)] = zero

    for g in range(NG):
        mem_copies[g].wait()
        buf = g % 2
        for i in range(CH):
            nb = g * GSZ + i * L
            flat = (nb + lane) * NB
            addr = jnp.zeros((L,), jnp.int32)
            for b in range(NB):
                fb = flat + b
                c = plsc.load_gather(
                    conn_v, [jnp.right_shift(fb, 7), jnp.bitwise_and(fb, 127)])
                w = plsc.load_gather(
                    proj_v, [jnp.right_shift(c, 7), jnp.bitwise_and(c, 127)])
                bit = plsc.load_gather(x_v, [w])
                addr = addr + bit * (1 << b)
            val = plsc.load_gather(rows_v.at[buf], [i * L + lane, addr])
            bits_v[g, pl.ds(i * L, L)] = jnp.where(val > 0.5, 1, 0).astype(jnp.int32)
        if g + 2 < NG:
            mem_copies[g + 2] = pltpu.async_copy(
                mem_hbm.at[sid, pl.ds(n0 + (g + 2) * GSZ, GSZ)],
                rows_v.at[buf], sem_mem)

    rowidx_v[...] = lane

    @pl.when(sid == 0)
    def _():
        pltpu.sync_copy(bits_v, shared)

    plsc.subcore_barrier()

    @pl.when(sid != 0)
    def _():
        pltpu.sync_copy(bits_v, shared.at[rowidx_v], add=True)

    plsc.subcore_barrier()

    @pl.when(sid < NG)
    def _():
        pltpu.sync_copy(shared, red_v)
        for k in range(CH):
            acc = red_v[sid, pl.ds(k * L, L)]
            outst_v[pl.ds(k * L, L)] = jnp.where(acc > R // 2, 1, 0).astype(jnp.int32)
        pltpu.sync_copy(outst_v, out_hbm.at[pl.ds(cid * HALF + sid * GSZ, GSZ)])


def kernel(x, projections, conn, memory):
    proj3 = projections.reshape(R, 32, 128)
    conn3 = conn.reshape(R, 128, 128)
    out = _ensemble_ram_sc(x, proj3, conn3, memory)
    return out.astype(jnp.uint8)
